# baseline (speedup 1.0000x reference)
import jax
import jax.numpy as jnp
from jax.experimental import pallas as pl
from jax.experimental.pallas import tpu as pltpu

_BN = 3200


def _chain_kernel(tok_ref, emb_ref, W1_ref, b1_ref, W2_ref, b2_ref,
                  out_ref, v_ref, acc_ref, sem):
    e = pl.program_id(0)
    ne = pl.num_programs(0)

    @pl.when(e == 0)
    def _init():
        tok = tok_ref[0, tok_ref.shape[1] - 1]
        cp = pltpu.make_async_copy(
            emb_ref.at[pl.ds(tok, 1), :], v_ref, sem)
        cp.start()
        cp.wait()

    t = jnp.maximum(
        jnp.dot(v_ref[...], W1_ref[0], preferred_element_type=jnp.float32)
        + b1_ref[0], 0.0)
    acc_ref[...] = (
        jnp.dot(t, W2_ref[0], preferred_element_type=jnp.float32) + b2_ref[0])
    v_ref[...] = acc_ref[...]

    @pl.when(e == ne - 1)
    def _emit():
        out_ref[...] = v_ref[...]


def _ntp_kernel(v_ref, W_ref, b_ref, out_ref):
    out_ref[...] = (
        jnp.dot(v_ref[...], W_ref[...], preferred_element_type=jnp.float32)
        + b_ref[...])


def kernel(X, emb, W1, b1, W2, b2, ntp_W, ntp_b):
    vocab, d = emb.shape
    nblocks, _, dff = W1.shape
    nexp = nblocks - 1

    tok = X.astype(jnp.int32)
    b1r = b1.reshape(nblocks, 1, dff)
    b2r = b2.reshape(nblocks, 1, d)

    v = pl.pallas_call(
        _chain_kernel,
        grid=(nexp,),
        in_specs=[
            pl.BlockSpec(memory_space=pltpu.SMEM),
            pl.BlockSpec(memory_space=pl.ANY),
            pl.BlockSpec((1, d, dff), lambda e: (e + 1, 0, 0)),
            pl.BlockSpec((1, 1, dff), lambda e: (e + 1, 0, 0)),
            pl.BlockSpec((1, dff, d), lambda e: (e + 1, 0, 0)),
            pl.BlockSpec((1, 1, d), lambda e: (e + 1, 0, 0)),
        ],
        out_specs=pl.BlockSpec((1, d), lambda e: (0, 0)),
        out_shape=jax.ShapeDtypeStruct((1, d), jnp.float32),
        scratch_shapes=[pltpu.VMEM((1, d), jnp.float32),
                        pltpu.VMEM((1, d), jnp.float32),
                        pltpu.SemaphoreType.DMA],
    )(tok, emb, W1, b1r, W2, b2r)

    nv = vocab // _BN
    logits = pl.pallas_call(
        _ntp_kernel,
        grid=(nv,),
        in_specs=[
            pl.BlockSpec((1, d), lambda j: (0, 0)),
            pl.BlockSpec((d, _BN), lambda j: (0, j)),
            pl.BlockSpec((1, _BN), lambda j: (0, j)),
        ],
        out_specs=pl.BlockSpec((1, _BN), lambda j: (0, j)),
        out_shape=jax.ShapeDtypeStruct((1, vocab), jnp.float32),
    )(v, ntp_W, ntp_b.reshape(1, vocab))
    return logits
